# baseline (speedup 1.0000x reference)
import functools

import jax
import jax.numpy as jnp
from jax import lax
from jax.experimental import pallas as pl
from jax.experimental.pallas import tpu as pltpu
from jax.experimental.pallas import tpu_sc as plsc

BATCH_SIZE = 16384
EMBED_DIM = 32

_info = plsc.get_sparse_core_info()
_NC, _NS, _LANES = _info.num_cores, _info.num_subcores, _info.num_lanes
_NW = _NC * _NS
_BPW = BATCH_SIZE // _NW
_CHUNK = 128
_NCHUNK = _BPW // _CHUNK
_GROUPS = _BPW // 16


def _vperm(x, idx):
    return lax.gather(
        x,
        idx[:, None],
        lax.GatherDimensionNumbers(
            offset_dims=(), collapsed_slice_dims=(0,), start_index_map=(0,)),
        (1,),
        mode=lax.GatherScatterMode.PROMISE_IN_BOUNDS,
    )


def _sc_body(uid_hbm, mid_hbm, uemb_hbm, memb_hbm, ub_hbm, mb_hbm, out_hbm,
             uid_v, mid_v, urows_v, mrows_v, ub_v, mb_v, out_v, sem):
    wid = lax.axis_index("s") * _NC + lax.axis_index("c")
    base = wid * _BPW

    pltpu.sync_copy(uid_hbm.at[pl.ds(base, _BPW)], uid_v)
    pltpu.sync_copy(mid_hbm.at[pl.ds(base, _BPW)], mid_v)

    copies = []
    for c in range(_NCHUNK):
        s = pl.ds(c * _CHUNK, _CHUNK)
        copies.append(pltpu.async_copy(uemb_hbm.at[uid_v.at[s]], urows_v.at[s], sem))
        copies.append(pltpu.async_copy(memb_hbm.at[mid_v.at[s]], mrows_v.at[s], sem))
        copies.append(pltpu.async_copy(ub_hbm.at[uid_v.at[s]], ub_v.at[s], sem))
        copies.append(pltpu.async_copy(mb_hbm.at[mid_v.at[s]], mb_v.at[s], sem))
    for cp in copies:
        cp.wait()

    iota = lax.iota(jnp.int32, _LANES)
    perms = [iota ^ sh for sh in (8, 4, 2, 1)]

    def group(g, carry):
        r0 = g * 16
        acc = jnp.zeros((_LANES,), jnp.float32)
        for k in range(16):
            r = r0 + k
            u0 = urows_v[r, pl.ds(0, 16)]
            u1 = urows_v[r, pl.ds(16, 16)]
            m0 = mrows_v[r, pl.ds(0, 16)]
            m1 = mrows_v[r, pl.ds(16, 16)]
            p = u0 * m0 + u1 * m1
            for pm in perms:
                p = p + _vperm(p, pm)
            acc = jnp.where(iota == k, p, acc)
        out_v[pl.ds(r0, 16)] = acc + ub_v[pl.ds(r0, 16)] + mb_v[pl.ds(r0, 16)]
        return carry

    lax.fori_loop(0, _GROUPS, group, 0)

    pltpu.sync_copy(out_v, out_hbm.at[pl.ds(base, _BPW)])


_sc_kernel = functools.partial(
    pl.kernel,
    out_type=jax.ShapeDtypeStruct((BATCH_SIZE,), jnp.float32),
    mesh=plsc.VectorSubcoreMesh(core_axis_name="c", subcore_axis_name="s"),
    compiler_params=pltpu.CompilerParams(use_tc_tiling_on_sc=False),
    scratch_types=[
        pltpu.VMEM((_BPW,), jnp.int32),
        pltpu.VMEM((_BPW,), jnp.int32),
        pltpu.VMEM((_BPW, EMBED_DIM), jnp.float32),
        pltpu.VMEM((_BPW, EMBED_DIM), jnp.float32),
        pltpu.VMEM((_BPW,), jnp.float32),
        pltpu.VMEM((_BPW,), jnp.float32),
        pltpu.VMEM((_BPW,), jnp.float32),
        pltpu.SemaphoreType.DMA,
    ],
)(_sc_body)


def kernel(inputs, user_emb, movie_emb, user_bias, movie_bias):
    idx = inputs.astype(jnp.int32)
    uid = idx[:, 0]
    mid = idx[:, 1]
    return _sc_kernel(uid, mid, user_emb, movie_emb,
                      user_bias.reshape(-1), movie_bias.reshape(-1))


# speedup vs baseline: 4.3127x; 4.3127x over previous
import functools

import jax
import jax.numpy as jnp
from jax import lax
from jax.experimental import pallas as pl
from jax.experimental.pallas import tpu as pltpu
from jax.experimental.pallas import tpu_sc as plsc

BATCH_SIZE = 16384
EMBED_DIM = 32

_info = plsc.get_sparse_core_info()
_NC, _NS, _LANES = _info.num_cores, _info.num_subcores, _info.num_lanes
_NW = _NC * _NS
_BPW = BATCH_SIZE // _NW
_GROUPS = _BPW // 16
_RING = 8


def _vperm(x, idx):
    return lax.gather(
        x,
        idx[:, None],
        lax.GatherDimensionNumbers(
            offset_dims=(), collapsed_slice_dims=(0,), start_index_map=(0,)),
        (1,),
        mode=lax.GatherScatterMode.PROMISE_IN_BOUNDS,
    )


def _sc_body(uid_hbm, mid_hbm, uembT_hbm, membT_hbm, ubT_hbm, mbT_hbm,
             out_hbm, uid_v, mid_v, ring_u, ring_m, ring_ub, ring_mb, out_v,
             sem):
    wid = lax.axis_index("s") * _NC + lax.axis_index("c")
    base = wid * _BPW

    pltpu.sync_copy(uid_hbm.at[pl.ds(base, _BPW)], uid_v)
    pltpu.sync_copy(mid_hbm.at[pl.ds(base, _BPW)], mid_v)

    def fire(cu, cm, slot):
        bu = pl.multiple_of(cu & jnp.int32(-128), 128)
        bm = pl.multiple_of(cm & jnp.int32(-128), 128)
        pltpu.async_copy(uembT_hbm.at[:, pl.ds(bu, 128)], ring_u.at[slot], sem)
        pltpu.async_copy(membT_hbm.at[:, pl.ds(bm, 128)], ring_m.at[slot], sem)
        pltpu.async_copy(ubT_hbm.at[:, pl.ds(bu, 128)], ring_ub.at[slot], sem)
        pltpu.async_copy(mbT_hbm.at[:, pl.ds(bm, 128)], ring_mb.at[slot], sem)

    cvec_u0 = uid_v[pl.ds(0, 16)]
    cvec_m0 = mid_v[pl.ds(0, 16)]
    for k in range(_RING):
        fire(cvec_u0[k], cvec_m0[k], k)

    iota = lax.iota(jnp.int32, _LANES)
    perms = [iota ^ sh for sh in (8, 4, 2, 1)]
    iota_lo = iota
    iota_hi = iota + 16
    zero16 = jnp.zeros((_LANES,), jnp.int32)

    def group(g, carry):
        b0 = g * 16
        gnext = jnp.minimum(g + 1, _GROUPS - 1)
        cvec_u = uid_v[pl.ds(b0, 16)]
        cvec_m = mid_v[pl.ds(b0, 16)]
        cnext_u = uid_v[pl.ds(gnext * 16, 16)]
        cnext_m = mid_v[pl.ds(gnext * 16, 16)]
        acc = jnp.zeros((_LANES,), jnp.float32)
        for k in range(16):
            slot = k % _RING
            pltpu.make_async_copy(
                uembT_hbm.at[:, pl.ds(0, 128)], ring_u.at[slot], sem).wait()
            pltpu.make_async_copy(
                membT_hbm.at[:, pl.ds(0, 128)], ring_m.at[slot], sem).wait()
            pltpu.make_async_copy(
                ubT_hbm.at[:, pl.ds(0, 128)], ring_ub.at[slot], sem).wait()
            pltpu.make_async_copy(
                mbT_hbm.at[:, pl.ds(0, 128)], ring_mb.at[slot], sem).wait()
            lu = jnp.full((_LANES,), cvec_u[k] & 127, jnp.int32)
            lm = jnp.full((_LANES,), cvec_m[k] & 127, jnp.int32)
            u0 = plsc.load_gather(ring_u.at[slot], [iota_lo, lu])
            u1 = plsc.load_gather(ring_u.at[slot], [iota_hi, lu])
            m0 = plsc.load_gather(ring_m.at[slot], [iota_lo, lm])
            m1 = plsc.load_gather(ring_m.at[slot], [iota_hi, lm])
            ubv = plsc.load_gather(ring_ub.at[slot], [zero16, lu])
            mbv = plsc.load_gather(ring_mb.at[slot], [zero16, lm])
            p = u0 * m0 + u1 * m1
            for pm in perms:
                p = p + _vperm(p, pm)
            acc = jnp.where(iota == k, p + ubv + mbv, acc)
            if k < 16 - _RING:
                fire(cvec_u[k + _RING], cvec_m[k + _RING], slot)
            else:
                @pl.when(g < _GROUPS - 1)
                def _():
                    fire(cnext_u[k + _RING - 16], cnext_m[k + _RING - 16], slot)
        out_v[pl.ds(b0, 16)] = acc
        return carry

    lax.fori_loop(0, _GROUPS, group, 0)

    pltpu.sync_copy(out_v, out_hbm.at[pl.ds(base, _BPW)])


_sc_kernel = functools.partial(
    pl.kernel,
    out_type=jax.ShapeDtypeStruct((BATCH_SIZE,), jnp.float32),
    mesh=plsc.VectorSubcoreMesh(core_axis_name="c", subcore_axis_name="s"),
    compiler_params=pltpu.CompilerParams(
        use_tc_tiling_on_sc=True, needs_layout_passes=False),
    scratch_types=[
        pltpu.VMEM((_BPW,), jnp.int32),
        pltpu.VMEM((_BPW,), jnp.int32),
        pltpu.VMEM((_RING, EMBED_DIM, 128), jnp.float32),
        pltpu.VMEM((_RING, EMBED_DIM, 128), jnp.float32),
        pltpu.VMEM((_RING, 1, 128), jnp.float32),
        pltpu.VMEM((_RING, 1, 128), jnp.float32),
        pltpu.VMEM((_BPW,), jnp.float32),
        pltpu.SemaphoreType.DMA,
    ],
)(_sc_body)


def kernel(inputs, user_emb, movie_emb, user_bias, movie_bias):
    idx = inputs.astype(jnp.int32)
    uid = idx[:, 0]
    mid = idx[:, 1]
    return _sc_kernel(uid, mid, user_emb.T, movie_emb.T,
                      user_bias.T, movie_bias.T)
